# Initial kernel scaffold; baseline (speedup 1.0000x reference)
import jax
import jax.numpy as jnp
from jax.experimental import pallas as pl


def kernel(v2d, vt, vn, vi, vti, index_img):
    raise NotImplementedError("write your pallas kernel here")


# speedup vs baseline: 54.8318x; 54.8318x over previous
import functools

import jax
import jax.numpy as jnp
from jax import lax
from jax.experimental import pallas as pl
from jax.experimental.pallas import tpu as pltpu
from jax.experimental.pallas import tpu_sc as plsc

_H, _W = 512, 512
_B, _V, _F, _VT = 4, 10000, 20000, 12000
_FPAD = 20480
_NW = 32
_REC = 32
_NPIX = _B * _H * _W
_PIX_PER_W = _NPIX // _NW
_CHUNK = 1024
_SUB = 128
_FACES_PER_W = (_B * _FPAD) // _NW
_FCHUNK = 512

_mesh = plsc.VectorSubcoreMesh(core_axis_name="c", subcore_axis_name="s")
_params = pltpu.CompilerParams(
    needs_layout_passes=False, use_tc_tiling_on_sc=False
)


def _eclamp(x):
    return jnp.where(x < 0, jnp.minimum(x, -1e-8), jnp.maximum(x, 1e-8))


def _splat(c):
    return jnp.full((16,), c, jnp.int32)


@functools.partial(
    pl.kernel,
    mesh=_mesh,
    compiler_params=_params,
    out_type=jax.ShapeDtypeStruct((_B * _FPAD * _REC,), jnp.float32),
    scratch_types=[
        pltpu.VMEM((_V * 3,), jnp.float32),
        pltpu.VMEM((_V * 3,), jnp.float32),
        pltpu.VMEM((_VT * 2,), jnp.float32),
        pltpu.VMEM((_FCHUNK * 3,), jnp.int32),
        pltpu.VMEM((_FCHUNK * 3,), jnp.int32),
        pltpu.VMEM((_FCHUNK * _REC,), jnp.float32),
    ],
)
def _face_table(v2d_hbm, vt_hbm, vn_hbm, vi_hbm, vti_hbm, tbl_hbm,
                v2d_v, vn_v, vt_v, vi_v, vti_v, rec_v):
    wid = lax.axis_index("s") * 2 + lax.axis_index("c")
    b = wid // 8
    f0 = (wid % 8) * _FACES_PER_W
    pltpu.sync_copy(v2d_hbm.at[b], v2d_v)
    pltpu.sync_copy(vn_hbm.at[b], vn_v)
    pltpu.sync_copy(vt_hbm, vt_v)
    iota = lax.iota(jnp.int32, 16)

    def chunk_body(ci, carry):
        fbase = f0 + ci * _FCHUNK
        pltpu.sync_copy(vi_hbm.at[pl.ds(fbase * 3, _FCHUNK * 3)], vi_v)
        pltpu.sync_copy(vti_hbm.at[pl.ds(fbase * 3, _FCHUNK * 3)], vti_v)

        def vec_body(mi, carry2):
            lrows = iota + mi * 16
            lrows3 = lrows * 3
            i0 = plsc.load_gather(vi_v, [lrows3])
            i1 = plsc.load_gather(vi_v, [lrows3 + 1])
            i2 = plsc.load_gather(vi_v, [lrows3 + 2])
            t0 = plsc.load_gather(vti_v, [lrows3])
            t1 = plsc.load_gather(vti_v, [lrows3 + 1])
            t2 = plsc.load_gather(vti_v, [lrows3 + 2])
            i03 = i0 * 3
            i13 = i1 * 3
            i23 = i2 * 3
            v0x = plsc.load_gather(v2d_v, [i03])
            v0y = plsc.load_gather(v2d_v, [i03 + 1])
            v0z = plsc.load_gather(v2d_v, [i03 + 2])
            v1x = plsc.load_gather(v2d_v, [i13])
            v1y = plsc.load_gather(v2d_v, [i13 + 1])
            v1z = plsc.load_gather(v2d_v, [i13 + 2])
            v2x = plsc.load_gather(v2d_v, [i23])
            v2y = plsc.load_gather(v2d_v, [i23 + 1])
            v2z = plsc.load_gather(v2d_v, [i23 + 2])
            e1x = v1x - v0x
            e1y = v1y - v0y
            e2x = v2x - v0x
            e2y = v2y - v0y
            det = e1x * e2y - e1y * e2x
            den = _eclamp(det)
            w0 = 1.0 / _eclamp(v0z)
            w1 = 1.0 / _eclamp(v1z)
            w2 = 1.0 / _eclamp(v2z)
            t02 = t0 * 2
            t12 = t1 * 2
            t22 = t2 * 2
            vals = [
                e1x, e1y, e2x, e2y, v0x, v0y, den, w0, w1, w2,
                plsc.load_gather(vt_v, [t02]),
                plsc.load_gather(vt_v, [t02 + 1]),
                plsc.load_gather(vt_v, [t12]),
                plsc.load_gather(vt_v, [t12 + 1]),
                plsc.load_gather(vt_v, [t22]),
                plsc.load_gather(vt_v, [t22 + 1]),
                plsc.load_gather(vn_v, [i03]),
                plsc.load_gather(vn_v, [i03 + 1]),
                plsc.load_gather(vn_v, [i03 + 2]),
                plsc.load_gather(vn_v, [i13]),
                plsc.load_gather(vn_v, [i13 + 1]),
                plsc.load_gather(vn_v, [i13 + 2]),
                plsc.load_gather(vn_v, [i23]),
                plsc.load_gather(vn_v, [i23 + 1]),
                plsc.load_gather(vn_v, [i23 + 2]),
            ]
            recbase = lrows * _REC
            for col, val in enumerate(vals):
                plsc.store_scatter(rec_v, [recbase + col], val)
            return carry2

        lax.fori_loop(0, _FCHUNK // 16, vec_body, 0)
        pltpu.sync_copy(
            rec_v, tbl_hbm.at[pl.ds((b * _FPAD + fbase) * _REC, _FCHUNK * _REC)]
        )
        return carry

    lax.fori_loop(0, _FACES_PER_W // _FCHUNK, chunk_body, 0)


@functools.partial(
    pl.kernel,
    mesh=_mesh,
    compiler_params=_params,
    out_type=(
        jax.ShapeDtypeStruct((_NPIX,), jnp.float32),
        jax.ShapeDtypeStruct((_NPIX * 3,), jnp.float32),
        jax.ShapeDtypeStruct((_NPIX * 2,), jnp.float32),
        jax.ShapeDtypeStruct((_NPIX * 3,), jnp.float32),
    ),
    scratch_types=[
        pltpu.VMEM((_CHUNK,), jnp.int32),
        pltpu.VMEM((_CHUNK // _SUB, _SUB), jnp.int32),
        pltpu.VMEM((_CHUNK, _REC), jnp.float32),
        pltpu.VMEM((_CHUNK,), jnp.float32),
        pltpu.VMEM((_CHUNK * 3,), jnp.float32),
        pltpu.VMEM((_CHUNK * 2,), jnp.float32),
        pltpu.VMEM((_CHUNK * 3,), jnp.float32),
        pltpu.SemaphoreType.DMA,
    ],
)
def _rasterize(tbl_hbm, idx_hbm, depth_hbm, bary_hbm, vto_hbm, vno_hbm,
               idx_raw, idx_adj, rows_v, dep_v, bar_v, vtv, vnv, sem):
    wid = lax.axis_index("s") * 2 + lax.axis_index("c")
    boff = (wid // 8) * _FPAD
    g0w = wid * _PIX_PER_W
    iota = lax.iota(jnp.int32, 16)

    def chunk_body(ci, carry):
        g0 = g0w + ci * _CHUNK
        pltpu.sync_copy(idx_hbm.at[pl.ds(g0, _CHUNK)], idx_raw)

        def adj_body(k, carry2):
            j = k // (_SUB // 16)
            m = k % (_SUB // 16)
            v = idx_raw[pl.ds(k * 16, 16)] + boff
            idx_adj[j, pl.ds(m * 16, 16)] = v
            return carry2

        lax.fori_loop(0, _CHUNK // 16, adj_body, 0)

        copies = [
            pltpu.async_copy(
                tbl_hbm.at[idx_adj.at[j]],
                rows_v.at[pl.ds(j * _SUB, _SUB)],
                sem,
            )
            for j in range(_CHUNK // _SUB)
        ]
        for c in copies:
            c.wait()

        def vec_body(mi, carry2):
            r = mi * 16
            rows = iota + r
            col = lambda cc: plsc.load_gather(rows_v, [rows, _splat(cc)])
            gv = iota + (g0 + r)
            x = (gv & 511).astype(jnp.float32)
            y = ((gv >> 9) & 511).astype(jnp.float32)
            e1x = col(0); e1y = col(1); e2x = col(2); e2y = col(3)
            v0x = col(4); v0y = col(5); den = col(6)
            w0 = col(7); w1 = col(8); w2 = col(9)
            px = x - v0x
            py = y - v0y
            l1 = (px * e2y - py * e2x) / den
            l2 = (py * e1x - px * e1y) / den
            l0 = 1.0 - l1 - l2
            u0 = w0 * l0
            u1 = w1 * l1
            u2 = w2 * l2
            zi = 1.0 / _eclamp(u0 + u1 + u2)
            q0 = u0 * zi
            q1 = u1 * zi
            q2 = u2 * zi
            dep_v[pl.ds(r, 16)] = zi
            rows3 = rows * 3
            rows2 = rows * 2
            plsc.store_scatter(bar_v, [rows3], q0)
            plsc.store_scatter(bar_v, [rows3 + 1], q1)
            plsc.store_scatter(bar_v, [rows3 + 2], q2)
            vtx = (col(10) * q0 + col(12) * q1 + col(14) * q2) * 2.0 - 1.0
            vty = (col(11) * q0 + col(13) * q1 + col(15) * q2) * 2.0 - 1.0
            plsc.store_scatter(vtv, [rows2], vtx)
            plsc.store_scatter(vtv, [rows2 + 1], vty)
            vnx = col(16) * q0 + col(19) * q1 + col(22) * q2
            vny = col(17) * q0 + col(20) * q1 + col(23) * q2
            vnz = col(18) * q0 + col(21) * q1 + col(24) * q2
            plsc.store_scatter(vnv, [rows3], vnx)
            plsc.store_scatter(vnv, [rows3 + 1], vny)
            plsc.store_scatter(vnv, [rows3 + 2], vnz)
            return carry2

        lax.fori_loop(0, _CHUNK // 16, vec_body, 0)
        pltpu.sync_copy(dep_v, depth_hbm.at[pl.ds(g0, _CHUNK)])
        pltpu.sync_copy(bar_v, bary_hbm.at[pl.ds(g0 * 3, _CHUNK * 3)])
        pltpu.sync_copy(vtv, vto_hbm.at[pl.ds(g0 * 2, _CHUNK * 2)])
        pltpu.sync_copy(vnv, vno_hbm.at[pl.ds(g0 * 3, _CHUNK * 3)])
        return carry

    lax.fori_loop(0, _PIX_PER_W // _CHUNK, chunk_body, 0)


def kernel(v2d, vt, vn, vi, vti, index_img):
    vi_p = jnp.pad(vi, ((0, _FPAD - _F), (0, 0))).reshape(-1)
    vti_p = jnp.pad(vti, ((0, _FPAD - _F), (0, 0))).reshape(-1)
    v2d_f = v2d.reshape(_B, _V * 3)
    vn_f = vn.reshape(_B, _V * 3)
    vt_f = vt.reshape(-1)
    tbl = _face_table(v2d_f, vt_f, vn_f, vi_p, vti_p)
    tbl2 = tbl.reshape(_B * _FPAD, _REC)
    idx_flat = index_img.reshape(-1)
    depth, bary, vto, vno = _rasterize(tbl2, idx_flat)
    return (
        depth.reshape(_B, _H, _W),
        bary.reshape(_B, _H, _W, 3),
        vto.reshape(_B, _H, _W, 2),
        vno.reshape(_B, _H, _W, 3),
    )


# speedup vs baseline: 199.9108x; 3.6459x over previous
import functools

import jax
import jax.numpy as jnp
from jax import lax
from jax.experimental import pallas as pl
from jax.experimental.pallas import tpu as pltpu
from jax.experimental.pallas import tpu_sc as plsc

_H, _W = 512, 512
_B, _V, _F, _VT = 4, 10000, 20000, 12000
_FPAD = 20480
_NW = 32
_REC = 32
_NPIX = _B * _H * _W
_CHUNK = 4096
_SUBPIX = 1024
_SUB = 128
_FACES_PER_W = (_B * _FPAD) // _NW
_FCHUNK = 512

_mesh = plsc.VectorSubcoreMesh(core_axis_name="c", subcore_axis_name="s")
_params = pltpu.CompilerParams(
    needs_layout_passes=False, use_tc_tiling_on_sc=False
)


def _eclamp(x):
    return jnp.where(x < 0, jnp.minimum(x, -1e-8), jnp.maximum(x, 1e-8))


def _splat(c):
    return jnp.full((16,), c, jnp.int32)


@functools.partial(
    pl.kernel,
    mesh=_mesh,
    compiler_params=_params,
    out_type=jax.ShapeDtypeStruct((_B * _FPAD * _REC,), jnp.float32),
    scratch_types=[
        pltpu.VMEM((_V * 3,), jnp.float32),
        pltpu.VMEM((_V * 3,), jnp.float32),
        pltpu.VMEM((_VT * 2,), jnp.float32),
        pltpu.VMEM((_FCHUNK * 3,), jnp.int32),
        pltpu.VMEM((_FCHUNK * 3,), jnp.int32),
        pltpu.VMEM((_FCHUNK * _REC,), jnp.float32),
    ],
)
def _face_table(v2d_hbm, vt_hbm, vn_hbm, vi_hbm, vti_hbm, tbl_hbm,
                v2d_v, vn_v, vt_v, vi_v, vti_v, rec_v):
    wid = lax.axis_index("s") * 2 + lax.axis_index("c")
    b = wid // 8
    f0 = (wid % 8) * _FACES_PER_W
    pltpu.sync_copy(v2d_hbm.at[b], v2d_v)
    pltpu.sync_copy(vn_hbm.at[b], vn_v)
    pltpu.sync_copy(vt_hbm, vt_v)
    iota = lax.iota(jnp.int32, 16)

    def chunk_body(ci, carry):
        fbase = jnp.minimum(f0 + ci * _FCHUNK, _F - _FCHUNK)
        pltpu.sync_copy(vi_hbm.at[pl.ds(fbase * 3, _FCHUNK * 3)], vi_v)
        pltpu.sync_copy(vti_hbm.at[pl.ds(fbase * 3, _FCHUNK * 3)], vti_v)

        def vec_body(mi, carry2):
            lrows = iota + mi * 16
            lrows3 = lrows * 3
            i0 = plsc.load_gather(vi_v, [lrows3])
            i1 = plsc.load_gather(vi_v, [lrows3 + 1])
            i2 = plsc.load_gather(vi_v, [lrows3 + 2])
            t0 = plsc.load_gather(vti_v, [lrows3])
            t1 = plsc.load_gather(vti_v, [lrows3 + 1])
            t2 = plsc.load_gather(vti_v, [lrows3 + 2])
            i03 = i0 * 3
            i13 = i1 * 3
            i23 = i2 * 3
            v0x = plsc.load_gather(v2d_v, [i03])
            v0y = plsc.load_gather(v2d_v, [i03 + 1])
            v0z = plsc.load_gather(v2d_v, [i03 + 2])
            v1x = plsc.load_gather(v2d_v, [i13])
            v1y = plsc.load_gather(v2d_v, [i13 + 1])
            v1z = plsc.load_gather(v2d_v, [i13 + 2])
            v2x = plsc.load_gather(v2d_v, [i23])
            v2y = plsc.load_gather(v2d_v, [i23 + 1])
            v2z = plsc.load_gather(v2d_v, [i23 + 2])
            e1x = v1x - v0x
            e1y = v1y - v0y
            e2x = v2x - v0x
            e2y = v2y - v0y
            det = e1x * e2y - e1y * e2x
            den = _eclamp(det)
            w0 = 1.0 / _eclamp(v0z)
            w1 = 1.0 / _eclamp(v1z)
            w2 = 1.0 / _eclamp(v2z)
            t02 = t0 * 2
            t12 = t1 * 2
            t22 = t2 * 2
            vals = [
                e1x, e1y, e2x, e2y, v0x, v0y, den, w0, w1, w2,
                plsc.load_gather(vt_v, [t02]),
                plsc.load_gather(vt_v, [t02 + 1]),
                plsc.load_gather(vt_v, [t12]),
                plsc.load_gather(vt_v, [t12 + 1]),
                plsc.load_gather(vt_v, [t22]),
                plsc.load_gather(vt_v, [t22 + 1]),
                plsc.load_gather(vn_v, [i03]),
                plsc.load_gather(vn_v, [i03 + 1]),
                plsc.load_gather(vn_v, [i03 + 2]),
                plsc.load_gather(vn_v, [i13]),
                plsc.load_gather(vn_v, [i13 + 1]),
                plsc.load_gather(vn_v, [i13 + 2]),
                plsc.load_gather(vn_v, [i23]),
                plsc.load_gather(vn_v, [i23 + 1]),
                plsc.load_gather(vn_v, [i23 + 2]),
            ]
            recbase = lrows * _REC
            for col, val in enumerate(vals):
                plsc.store_scatter(rec_v, [recbase + col], val)
            return carry2

        lax.fori_loop(0, _FCHUNK // 16, vec_body, 0)
        pltpu.sync_copy(
            rec_v, tbl_hbm.at[pl.ds((b * _FPAD + fbase) * _REC, _FCHUNK * _REC)]
        )
        return carry

    lax.fori_loop(0, _FACES_PER_W // _FCHUNK, chunk_body, 0)


@functools.partial(
    pl.kernel,
    mesh=_mesh,
    compiler_params=_params,
    out_type=(
        jax.ShapeDtypeStruct((_NPIX,), jnp.float32),
        jax.ShapeDtypeStruct((_NPIX * 3,), jnp.float32),
        jax.ShapeDtypeStruct((_NPIX * 2,), jnp.float32),
        jax.ShapeDtypeStruct((_NPIX * 3,), jnp.float32),
    ),
    scratch_types=[
        pltpu.VMEM((_CHUNK,), jnp.int32),
        pltpu.VMEM((_CHUNK // _SUB, _SUB), jnp.int32),
        pltpu.VMEM((_SUBPIX, _REC), jnp.float32),
        pltpu.VMEM((_SUBPIX, _REC), jnp.float32),
        pltpu.VMEM((_CHUNK,), jnp.float32),
        pltpu.VMEM((_CHUNK,), jnp.float32),
        pltpu.VMEM((_CHUNK,), jnp.float32),
        pltpu.VMEM((_CHUNK,), jnp.float32),
        pltpu.VMEM((_CHUNK,), jnp.float32),
        pltpu.VMEM((_CHUNK,), jnp.float32),
        pltpu.VMEM((_CHUNK,), jnp.float32),
        pltpu.VMEM((_CHUNK * 2,), jnp.float32),
        pltpu.SemaphoreType.DMA,
        pltpu.SemaphoreType.DMA,
        pltpu.SemaphoreType.DMA,
    ],
)
def _rasterize(tbl_hbm, idx_hbm, depth_hbm, bary_hbm, vto_hbm, vno_hbm,
               idx_raw, idx_adj, rows_a, rows_b, dep_v, b0_v, b1_v, b2_v,
               n0_v, n1_v, n2_v, vt_v, sema, semb, semo):
    wid = lax.axis_index("s") * 2 + lax.axis_index("c")
    b = wid // 8
    boff = b * _FPAD
    y0 = (wid % 8) * 64
    iota = lax.iota(jnp.int32, 16)
    rows_bufs = (rows_a, rows_b)
    sems = (sema, semb)

    def out_copies(rb):
        dep_off = (b * 64 + rb) * _CHUNK
        yc = rb * 8
        return [
            (dep_v, depth_hbm, dep_off, _CHUNK),
            (b0_v, bary_hbm, ((b * 3 + 0) * 64 + rb) * _CHUNK, _CHUNK),
            (b1_v, bary_hbm, ((b * 3 + 1) * 64 + rb) * _CHUNK, _CHUNK),
            (b2_v, bary_hbm, ((b * 3 + 2) * 64 + rb) * _CHUNK, _CHUNK),
            (n0_v, vno_hbm, ((b * 3 + 0) * 64 + rb) * _CHUNK, _CHUNK),
            (n1_v, vno_hbm, ((b * 3 + 1) * 64 + rb) * _CHUNK, _CHUNK),
            (n2_v, vno_hbm, ((b * 3 + 2) * 64 + rb) * _CHUNK, _CHUNK),
            (vt_v, vto_hbm, (b * 512 + yc) * 1024, _CHUNK * 2),
        ]

    def chunk_body(ci, carry):
        yc = y0 + ci * 8
        rb = yc >> 3
        g0 = b * (_H * _W) + yc * _W
        pltpu.sync_copy(idx_hbm.at[pl.ds(g0, _CHUNK)], idx_raw)

        def adj_body(k, carry2):
            j = k // (_SUB // 16)
            m = k % (_SUB // 16)
            v = idx_raw[pl.ds(k * 16, 16)] + boff
            idx_adj[j, pl.ds(m * 16, 16)] = v
            return carry2

        lax.fori_loop(0, _CHUNK // 16, adj_body, 0)

        def fire(s):
            buf = rows_bufs[s % 2]
            sem = sems[s % 2]
            return [
                pltpu.async_copy(
                    tbl_hbm.at[idx_adj.at[s * 8 + j]],
                    buf.at[pl.ds(j * _SUB, _SUB)],
                    sem,
                )
                for j in range(_SUBPIX // _SUB)
            ]

        inflight = fire(0)

        @pl.when(ci > 0)
        def _():
            for src, dst, off, n in out_copies(rb):
                pltpu.make_async_copy(src, dst.at[pl.ds(off, n)], semo).wait()

        for s in range(_CHUNK // _SUBPIX):
            nxt = fire(s + 1) if s + 1 < _CHUNK // _SUBPIX else []
            for h in inflight:
                h.wait()
            inflight = nxt
            buf = rows_bufs[s % 2]
            l0 = s * _SUBPIX

            def vec_body(mi, carry2, buf=buf, l0=l0):
                m16 = mi * 16
                lrow = iota + m16
                col = lambda cc: plsc.load_gather(buf, [lrow, _splat(cc)])
                l = l0 + m16
                lx = l & 511
                x = (iota + lx).astype(jnp.float32)
                y = (yc + (l >> 9)).astype(jnp.float32)
                e1x = col(0); e1y = col(1); e2x = col(2); e2y = col(3)
                v0x = col(4); v0y = col(5); den = col(6)
                w0 = col(7); w1 = col(8); w2 = col(9)
                px = x - v0x
                py = y - v0y
                l1 = (px * e2y - py * e2x) / den
                l2 = (py * e1x - px * e1y) / den
                lam0 = 1.0 - l1 - l2
                u0 = w0 * lam0
                u1 = w1 * l1
                u2 = w2 * l2
                zi = 1.0 / _eclamp(u0 + u1 + u2)
                q0 = u0 * zi
                q1 = u1 * zi
                q2 = u2 * zi
                off = (lx >> 7) * 1024 + (l >> 9) * 128 + (l & 127)
                vtoff = (l >> 9) * 1024 + (lx >> 7) * 256 + (l & 127)
                dep_v[pl.ds(off, 16)] = zi
                b0_v[pl.ds(off, 16)] = q0
                b1_v[pl.ds(off, 16)] = q1
                b2_v[pl.ds(off, 16)] = q2
                vtx = (col(10) * q0 + col(12) * q1 + col(14) * q2) * 2.0 - 1.0
                vty = (col(11) * q0 + col(13) * q1 + col(15) * q2) * 2.0 - 1.0
                vt_v[pl.ds(vtoff, 16)] = vtx
                vt_v[pl.ds(vtoff + 128, 16)] = vty
                n0_v[pl.ds(off, 16)] = col(16) * q0 + col(19) * q1 + col(22) * q2
                n1_v[pl.ds(off, 16)] = col(17) * q0 + col(20) * q1 + col(23) * q2
                n2_v[pl.ds(off, 16)] = col(18) * q0 + col(21) * q1 + col(24) * q2
                return carry2

            lax.fori_loop(0, _SUBPIX // 16, vec_body, 0)

        for src, dst, off, n in out_copies(rb):
            pltpu.async_copy(src, dst.at[pl.ds(off, n)], semo)
        return carry

    lax.fori_loop(0, 8, chunk_body, 0)

    for src, dst, off, n in out_copies((y0 >> 3) + 7):
        pltpu.make_async_copy(src, dst.at[pl.ds(off, n)], semo).wait()


def kernel(v2d, vt, vn, vi, vti, index_img):
    vi_f = vi.reshape(-1)
    vti_f = vti.reshape(-1)
    v2d_f = v2d.reshape(_B, _V * 3)
    vn_f = vn.reshape(_B, _V * 3)
    vt_f = vt.reshape(-1)
    tbl = _face_table(v2d_f, vt_f, vn_f, vi_f, vti_f)
    tbl2 = tbl.reshape(_B * _FPAD, _REC)
    idx_flat = index_img.reshape(-1)
    depth, bary, vto, vno = _rasterize(tbl2, idx_flat)
    depth = depth.reshape(_B, 64, 4, 8, 128).transpose(0, 1, 3, 2, 4)
    depth = depth.reshape(_B, _H, _W)
    bary = bary.reshape(_B, 3, 64, 4, 8, 128).transpose(0, 2, 4, 3, 5, 1)
    bary = bary.reshape(_B, _H, _W, 3)
    vno = vno.reshape(_B, 3, 64, 4, 8, 128).transpose(0, 2, 4, 3, 5, 1)
    vno = vno.reshape(_B, _H, _W, 3)
    vto = vto.reshape(_B, _H, 4, 2, 128).transpose(0, 1, 2, 4, 3)
    vto = vto.reshape(_B, _H, _W, 2)
    return depth, bary, vto, vno


# speedup vs baseline: 205.6067x; 1.0285x over previous
import functools

import jax
import jax.numpy as jnp
from jax import lax
from jax.experimental import pallas as pl
from jax.experimental.pallas import tpu as pltpu
from jax.experimental.pallas import tpu_sc as plsc

_H, _W = 512, 512
_B, _V, _F, _VT = 4, 10000, 20000, 12000
_FPAD = 20480
_NW = 32
_REC = 32
_NPIX = _B * _H * _W
_CHUNK = 4096
_SUBPIX = 1024
_SUB = 128
_NSUB = _CHUNK // _SUBPIX
_FACES_PER_W = (_B * _FPAD) // _NW
_FCHUNK = 512

_mesh = plsc.VectorSubcoreMesh(core_axis_name="c", subcore_axis_name="s")
_params = pltpu.CompilerParams(
    needs_layout_passes=False, use_tc_tiling_on_sc=False
)


def _eclamp(x):
    return jnp.where(x < 0, jnp.minimum(x, -1e-8), jnp.maximum(x, 1e-8))


def _splat(c):
    return jnp.full((16,), c, jnp.int32)


@functools.partial(
    pl.kernel,
    mesh=_mesh,
    compiler_params=_params,
    out_type=jax.ShapeDtypeStruct((_B * _FPAD * _REC,), jnp.float32),
    scratch_types=[
        pltpu.VMEM((_V * 3,), jnp.float32),
        pltpu.VMEM((_V * 3,), jnp.float32),
        pltpu.VMEM((_VT * 2,), jnp.float32),
        pltpu.VMEM((_FCHUNK * 3,), jnp.int32),
        pltpu.VMEM((_FCHUNK * 3,), jnp.int32),
        pltpu.VMEM((_FCHUNK * _REC,), jnp.float32),
        pltpu.SemaphoreType.DMA,
    ],
)
def _face_table(v2d_hbm, vt_hbm, vn_hbm, vi_hbm, vti_hbm, tbl_hbm,
                v2d_v, vn_v, vt_v, vi_v, vti_v, rec_v, sem):
    wid = lax.axis_index("s") * 2 + lax.axis_index("c")
    b = wid // 8
    f0 = (wid % 8) * _FACES_PER_W
    tbl_loads = [
        pltpu.async_copy(v2d_hbm.at[b], v2d_v, sem),
        pltpu.async_copy(vn_hbm.at[b], vn_v, sem),
        pltpu.async_copy(vt_hbm, vt_v, sem),
    ]
    for h in tbl_loads:
        h.wait()
    iota = lax.iota(jnp.int32, 16)

    def chunk_body(ci, carry):
        fbase = jnp.minimum(f0 + ci * _FCHUNK, _F - _FCHUNK)
        idx_loads = [
            pltpu.async_copy(
                vi_hbm.at[pl.ds(fbase * 3, _FCHUNK * 3)], vi_v, sem
            ),
            pltpu.async_copy(
                vti_hbm.at[pl.ds(fbase * 3, _FCHUNK * 3)], vti_v, sem
            ),
        ]
        for h in idx_loads:
            h.wait()

        def vec_body(mi, carry2):
            lrows = iota + mi * 16
            lrows3 = lrows * 3
            i0 = plsc.load_gather(vi_v, [lrows3])
            i1 = plsc.load_gather(vi_v, [lrows3 + 1])
            i2 = plsc.load_gather(vi_v, [lrows3 + 2])
            t0 = plsc.load_gather(vti_v, [lrows3])
            t1 = plsc.load_gather(vti_v, [lrows3 + 1])
            t2 = plsc.load_gather(vti_v, [lrows3 + 2])
            i03 = i0 * 3
            i13 = i1 * 3
            i23 = i2 * 3
            v0x = plsc.load_gather(v2d_v, [i03])
            v0y = plsc.load_gather(v2d_v, [i03 + 1])
            v0z = plsc.load_gather(v2d_v, [i03 + 2])
            v1x = plsc.load_gather(v2d_v, [i13])
            v1y = plsc.load_gather(v2d_v, [i13 + 1])
            v1z = plsc.load_gather(v2d_v, [i13 + 2])
            v2x = plsc.load_gather(v2d_v, [i23])
            v2y = plsc.load_gather(v2d_v, [i23 + 1])
            v2z = plsc.load_gather(v2d_v, [i23 + 2])
            e1x = v1x - v0x
            e1y = v1y - v0y
            e2x = v2x - v0x
            e2y = v2y - v0y
            det = e1x * e2y - e1y * e2x
            den = _eclamp(det)
            w0 = 1.0 / _eclamp(v0z)
            w1 = 1.0 / _eclamp(v1z)
            w2 = 1.0 / _eclamp(v2z)
            t02 = t0 * 2
            t12 = t1 * 2
            t22 = t2 * 2
            vals = [
                e1x, e1y, e2x, e2y, v0x, v0y, den, w0, w1, w2,
                plsc.load_gather(vt_v, [t02]),
                plsc.load_gather(vt_v, [t02 + 1]),
                plsc.load_gather(vt_v, [t12]),
                plsc.load_gather(vt_v, [t12 + 1]),
                plsc.load_gather(vt_v, [t22]),
                plsc.load_gather(vt_v, [t22 + 1]),
                plsc.load_gather(vn_v, [i03]),
                plsc.load_gather(vn_v, [i03 + 1]),
                plsc.load_gather(vn_v, [i03 + 2]),
                plsc.load_gather(vn_v, [i13]),
                plsc.load_gather(vn_v, [i13 + 1]),
                plsc.load_gather(vn_v, [i13 + 2]),
                plsc.load_gather(vn_v, [i23]),
                plsc.load_gather(vn_v, [i23 + 1]),
                plsc.load_gather(vn_v, [i23 + 2]),
            ]
            recbase = lrows * _REC
            for col, val in enumerate(vals):
                plsc.store_scatter(rec_v, [recbase + col], val)
            return carry2

        lax.fori_loop(0, _FCHUNK // 16, vec_body, 0)
        pltpu.sync_copy(
            rec_v, tbl_hbm.at[pl.ds((b * _FPAD + fbase) * _REC, _FCHUNK * _REC)]
        )
        return carry

    lax.fori_loop(0, _FACES_PER_W // _FCHUNK, chunk_body, 0)


@functools.partial(
    pl.kernel,
    mesh=_mesh,
    compiler_params=_params,
    out_type=(
        jax.ShapeDtypeStruct((_NPIX,), jnp.float32),
        jax.ShapeDtypeStruct((_NPIX * 3,), jnp.float32),
        jax.ShapeDtypeStruct((_NPIX * 2,), jnp.float32),
        jax.ShapeDtypeStruct((_NPIX * 3,), jnp.float32),
    ),
    scratch_types=[
        pltpu.VMEM((_CHUNK,), jnp.int32),
        pltpu.VMEM((_CHUNK // _SUB, _SUB), jnp.int32),
        pltpu.VMEM((_SUBPIX, _REC), jnp.float32),
        pltpu.VMEM((_SUBPIX, _REC), jnp.float32),
        pltpu.VMEM((_CHUNK,), jnp.float32),
        pltpu.VMEM((_CHUNK,), jnp.float32),
        pltpu.VMEM((_CHUNK,), jnp.float32),
        pltpu.VMEM((_CHUNK,), jnp.float32),
        pltpu.VMEM((_CHUNK,), jnp.float32),
        pltpu.VMEM((_CHUNK,), jnp.float32),
        pltpu.VMEM((_CHUNK,), jnp.float32),
        pltpu.VMEM((_CHUNK * 2,), jnp.float32),
        pltpu.SemaphoreType.DMA,
        pltpu.SemaphoreType.DMA,
        pltpu.SemaphoreType.DMA,
    ],
)
def _rasterize(tbl_hbm, idx_hbm, depth_hbm, bary_hbm, vto_hbm, vno_hbm,
               idx_raw, idx_adj, rows_a, rows_b, dep_v, b0_v, b1_v, b2_v,
               n0_v, n1_v, n2_v, vt_v, sema, semb, semo):
    wid = lax.axis_index("s") * 2 + lax.axis_index("c")
    b = wid // 8
    boff = b * _FPAD
    y0 = (wid % 8) * 64
    rb0 = b * 64 + (y0 >> 3)
    iota = lax.iota(jnp.int32, 16)
    rows_bufs = (rows_a, rows_b)
    sems = (sema, semb)

    def load_adjust(ci):
        pltpu.sync_copy(idx_hbm.at[pl.ds((rb0 + ci) * _CHUNK, _CHUNK)], idx_raw)

        def adj_body(k, carry2):
            j = k // (_SUB // 16)
            m = k % (_SUB // 16)
            v = idx_raw[pl.ds(k * 16, 16)] + boff
            idx_adj[j, pl.ds(m * 16, 16)] = v
            return carry2

        lax.fori_loop(0, _CHUNK // 16, adj_body, 0)

    def fire(s):
        buf = rows_bufs[s % 2]
        sem = sems[s % 2]
        return [
            pltpu.async_copy(
                tbl_hbm.at[idx_adj.at[s * 8 + j]],
                buf.at[pl.ds(j * _SUB, _SUB)],
                sem,
            )
            for j in range(_SUBPIX // _SUB)
        ]

    def drain_gather(s):
        buf = rows_bufs[s % 2]
        sem = sems[s % 2]
        for j in range(_SUBPIX // _SUB):
            pltpu.make_async_copy(
                tbl_hbm.at[idx_adj.at[s * 8 + j]],
                buf.at[pl.ds(j * _SUB, _SUB)],
                sem,
            ).wait()

    def out_copies(rb):
        return [
            (dep_v, depth_hbm, rb * _CHUNK, _CHUNK),
            (b0_v, bary_hbm, ((b * 2) * 64 + rb) * _CHUNK, _CHUNK),
            (b1_v, bary_hbm, ((b * 2 + 1) * 64 + rb) * _CHUNK, _CHUNK),
            (b2_v, bary_hbm, ((b * 2 + 2) * 64 + rb) * _CHUNK, _CHUNK),
            (n0_v, vno_hbm, ((b * 2) * 64 + rb) * _CHUNK, _CHUNK),
            (n1_v, vno_hbm, ((b * 2 + 1) * 64 + rb) * _CHUNK, _CHUNK),
            (n2_v, vno_hbm, ((b * 2 + 2) * 64 + rb) * _CHUNK, _CHUNK),
            (vt_v, vto_hbm, rb * _CHUNK * 2, _CHUNK * 2),
        ]

    load_adjust(0)
    fire(0)

    def chunk_body(ci, carry):
        yc = y0 + ci * 8
        rb = rb0 + ci

        prev_handles = None
        for s in range(_NSUB):
            nxt = fire(s + 1) if s + 1 < _NSUB else None
            if s == 0:
                drain_gather(0)

                @pl.when(ci > 0)
                def _():
                    for src, dst, off, n in out_copies(rb):
                        pltpu.make_async_copy(
                            src, dst.at[pl.ds(off, n)], semo
                        ).wait()
            else:
                for h in prev_handles:
                    h.wait()
            if s == _NSUB - 1:
                @pl.when(ci < 7)
                def _():
                    load_adjust(ci + 1)
                    fire(0)
            prev_handles = nxt
            buf = rows_bufs[s % 2]
            l0 = s * _SUBPIX

            def vec_body(mi, carry2, buf=buf, l0=l0):
                for half in range(2):
                    m16 = mi * 32 + half * 16
                    lrow = iota + m16
                    col = lambda cc: plsc.load_gather(buf, [lrow, _splat(cc)])
                    l = l0 + m16
                    xb = ((l >> 10) << 7) | (l & 127)
                    x = (iota + xb).astype(jnp.float32)
                    y = (yc + ((l >> 7) & 7)).astype(jnp.float32)
                    e1x = col(0); e1y = col(1); e2x = col(2); e2y = col(3)
                    v0x = col(4); v0y = col(5); den = col(6)
                    w0 = col(7); w1 = col(8); w2 = col(9)
                    px = x - v0x
                    py = y - v0y
                    l1 = (px * e2y - py * e2x) / den
                    l2 = (py * e1x - px * e1y) / den
                    lam0 = 1.0 - l1 - l2
                    u0 = w0 * lam0
                    u1 = w1 * l1
                    u2 = w2 * l2
                    zi = 1.0 / _eclamp(u0 + u1 + u2)
                    q0 = u0 * zi
                    q1 = u1 * zi
                    q2 = u2 * zi
                    vtoff = ((l >> 7) & 7) * 1024 + (l >> 10) * 256 + (l & 127)
                    dep_v[pl.ds(l, 16)] = zi
                    b0_v[pl.ds(l, 16)] = q0
                    b1_v[pl.ds(l, 16)] = q1
                    b2_v[pl.ds(l, 16)] = q2
                    vtx = (col(10) * q0 + col(12) * q1 + col(14) * q2) * 2.0 - 1.0
                    vty = (col(11) * q0 + col(13) * q1 + col(15) * q2) * 2.0 - 1.0
                    vt_v[pl.ds(vtoff, 16)] = vtx
                    vt_v[pl.ds(vtoff + 128, 16)] = vty
                    n0_v[pl.ds(l, 16)] = col(16) * q0 + col(19) * q1 + col(22) * q2
                    n1_v[pl.ds(l, 16)] = col(17) * q0 + col(20) * q1 + col(23) * q2
                    n2_v[pl.ds(l, 16)] = col(18) * q0 + col(21) * q1 + col(24) * q2
                return carry2

            lax.fori_loop(0, _SUBPIX // 32, vec_body, 0)

        for src, dst, off, n in out_copies(rb):
            pltpu.async_copy(src, dst.at[pl.ds(off, n)], semo)
        return carry

    lax.fori_loop(0, 8, chunk_body, 0)

    for src, dst, off, n in out_copies(rb0 + 7):
        pltpu.make_async_copy(src, dst.at[pl.ds(off, n)], semo).wait()


def kernel(v2d, vt, vn, vi, vti, index_img):
    vi_f = vi.reshape(-1)
    vti_f = vti.reshape(-1)
    v2d_f = v2d.reshape(_B, _V * 3)
    vn_f = vn.reshape(_B, _V * 3)
    vt_f = vt.reshape(-1)
    tbl = _face_table(v2d_f, vt_f, vn_f, vi_f, vti_f)
    tbl2 = tbl.reshape(_B * _FPAD, _REC)
    idx_tiled = index_img.reshape(_B, 64, 8, 4, 128)
    idx_tiled = idx_tiled.transpose(0, 1, 3, 2, 4).reshape(-1)
    depth, bary, vto, vno = _rasterize(tbl2, idx_tiled)
    depth = depth.reshape(_B, 64, 4, 8, 128).transpose(0, 1, 3, 2, 4)
    depth = depth.reshape(_B, _H, _W)
    bary = bary.reshape(_B, 3, 64, 4, 8, 128).transpose(0, 2, 4, 3, 5, 1)
    bary = bary.reshape(_B, _H, _W, 3)
    vno = vno.reshape(_B, 3, 64, 4, 8, 128).transpose(0, 2, 4, 3, 5, 1)
    vno = vno.reshape(_B, _H, _W, 3)
    vto = vto.reshape(_B, _H, 4, 2, 128).transpose(0, 1, 2, 4, 3)
    vto = vto.reshape(_B, _H, _W, 2)
    return depth, bary, vto, vno


# speedup vs baseline: 366.9220x; 1.7846x over previous
import functools

import jax
import jax.numpy as jnp
from jax import lax
from jax.experimental import pallas as pl
from jax.experimental.pallas import tpu as pltpu
from jax.experimental.pallas import tpu_sc as plsc

_H, _W = 512, 512
_B, _V, _F, _VT = 4, 10000, 20000, 12000
_NW = 32
_NPIX = _B * _H * _W
_CHUNK = 4096

_mesh = plsc.VectorSubcoreMesh(core_axis_name="c", subcore_axis_name="s")
_params = pltpu.CompilerParams(
    needs_layout_passes=False, use_tc_tiling_on_sc=False
)


def _eclamp(x):
    return jnp.where(x < 0, jnp.minimum(x, -1e-8), jnp.maximum(x, 1e-8))


def _worker():
    wid = lax.axis_index("s") * 2 + lax.axis_index("c")
    b = wid // 8
    y0 = (wid % 8) * 64
    rb0 = b * 64 + (y0 >> 3)
    return b, y0, rb0


def _plane_off(b, rb, c):
    return ((b * 2 + c) * 64 + rb) * _CHUNK


@functools.partial(
    pl.kernel,
    mesh=_mesh,
    compiler_params=_params,
    out_type=(
        jax.ShapeDtypeStruct((_NPIX,), jnp.float32),
        jax.ShapeDtypeStruct((_NPIX * 3,), jnp.float32),
    ),
    scratch_types=[
        pltpu.VMEM((_V * 3,), jnp.float32),
        pltpu.VMEM((_F * 3,), jnp.int32),
        pltpu.VMEM((_CHUNK,), jnp.int32),
        pltpu.VMEM((_CHUNK,), jnp.float32),
        pltpu.VMEM((_CHUNK,), jnp.float32),
        pltpu.VMEM((_CHUNK,), jnp.float32),
        pltpu.VMEM((_CHUNK,), jnp.float32),
        pltpu.SemaphoreType.DMA,
        pltpu.SemaphoreType.DMA,
    ],
)
def _geom(v2d_hbm, vi_hbm, idx_hbm, depth_hbm, bary_hbm,
          v2d_v, vi_v, idx_v, dep_v, q0_v, q1_v, q2_v, semt, semo):
    b, y0, rb0 = _worker()
    loads = [
        pltpu.async_copy(v2d_hbm.at[b], v2d_v, semt),
        pltpu.async_copy(vi_hbm, vi_v, semt),
    ]
    for h in loads:
        h.wait()
    iota = lax.iota(jnp.int32, 16)

    def outs(rb):
        return [
            (dep_v, depth_hbm, rb * _CHUNK),
            (q0_v, bary_hbm, _plane_off(b, rb, 0)),
            (q1_v, bary_hbm, _plane_off(b, rb, 1)),
            (q2_v, bary_hbm, _plane_off(b, rb, 2)),
        ]

    def chunk_body(ci, carry):
        yc = y0 + ci * 8
        rb = rb0 + ci
        pltpu.sync_copy(idx_hbm.at[pl.ds(rb * _CHUNK, _CHUNK)], idx_v)

        @pl.when(ci > 0)
        def _():
            for src, dst, off in outs(rb):
                pltpu.make_async_copy(src, dst.at[pl.ds(off, _CHUNK)], semo).wait()

        def vec_body(mi, carry2):
            for half in range(2):
                l = mi * 32 + half * 16
                f3 = idx_v[pl.ds(l, 16)] * 3
                i0 = plsc.load_gather(vi_v, [f3]) * 3
                i1 = plsc.load_gather(vi_v, [f3 + 1]) * 3
                i2 = plsc.load_gather(vi_v, [f3 + 2]) * 3
                v0x = plsc.load_gather(v2d_v, [i0])
                v0y = plsc.load_gather(v2d_v, [i0 + 1])
                v0z = plsc.load_gather(v2d_v, [i0 + 2])
                v1x = plsc.load_gather(v2d_v, [i1])
                v1y = plsc.load_gather(v2d_v, [i1 + 1])
                v1z = plsc.load_gather(v2d_v, [i1 + 2])
                v2x = plsc.load_gather(v2d_v, [i2])
                v2y = plsc.load_gather(v2d_v, [i2 + 1])
                v2z = plsc.load_gather(v2d_v, [i2 + 2])
                xb = ((l >> 10) << 7) | (l & 127)
                x = (iota + xb).astype(jnp.float32)
                y = (yc + ((l >> 7) & 7)).astype(jnp.float32)
                e1x = v1x - v0x
                e1y = v1y - v0y
                e2x = v2x - v0x
                e2y = v2y - v0y
                den = _eclamp(e1x * e2y - e1y * e2x)
                w0 = 1.0 / _eclamp(v0z)
                w1 = 1.0 / _eclamp(v1z)
                w2 = 1.0 / _eclamp(v2z)
                px = x - v0x
                py = y - v0y
                l1 = (px * e2y - py * e2x) / den
                l2 = (py * e1x - px * e1y) / den
                lam0 = 1.0 - l1 - l2
                u0 = w0 * lam0
                u1 = w1 * l1
                u2 = w2 * l2
                zi = 1.0 / _eclamp(u0 + u1 + u2)
                dep_v[pl.ds(l, 16)] = zi
                q0_v[pl.ds(l, 16)] = u0 * zi
                q1_v[pl.ds(l, 16)] = u1 * zi
                q2_v[pl.ds(l, 16)] = u2 * zi
            return carry2

        lax.fori_loop(0, _CHUNK // 32, vec_body, 0)
        for src, dst, off in outs(rb):
            pltpu.async_copy(src, dst.at[pl.ds(off, _CHUNK)], semo)
        return carry

    lax.fori_loop(0, 8, chunk_body, 0)
    for src, dst, off in outs(rb0 + 7):
        pltpu.make_async_copy(src, dst.at[pl.ds(off, _CHUNK)], semo).wait()


@functools.partial(
    pl.kernel,
    mesh=_mesh,
    compiler_params=_params,
    out_type=jax.ShapeDtypeStruct((_NPIX * 2,), jnp.float32),
    scratch_types=[
        pltpu.VMEM((_VT * 2,), jnp.float32),
        pltpu.VMEM((_F * 3,), jnp.int32),
        pltpu.VMEM((_CHUNK,), jnp.int32),
        pltpu.VMEM((_CHUNK,), jnp.float32),
        pltpu.VMEM((_CHUNK,), jnp.float32),
        pltpu.VMEM((_CHUNK,), jnp.float32),
        pltpu.VMEM((_CHUNK * 2,), jnp.float32),
        pltpu.SemaphoreType.DMA,
        pltpu.SemaphoreType.DMA,
        pltpu.SemaphoreType.DMA,
    ],
)
def _texco(vt_hbm, vti_hbm, idx_hbm, bary_hbm, vto_hbm,
           vt_v, vti_v, idx_v, q0_v, q1_v, q2_v, out_v, semt, semi, semo):
    b, y0, rb0 = _worker()
    loads = [
        pltpu.async_copy(vt_hbm, vt_v, semt),
        pltpu.async_copy(vti_hbm, vti_v, semt),
    ]
    for h in loads:
        h.wait()

    def chunk_body(ci, carry):
        rb = rb0 + ci
        ins = [
            pltpu.async_copy(idx_hbm.at[pl.ds(rb * _CHUNK, _CHUNK)], idx_v, semi),
            pltpu.async_copy(
                bary_hbm.at[pl.ds(_plane_off(b, rb, 0), _CHUNK)], q0_v, semi),
            pltpu.async_copy(
                bary_hbm.at[pl.ds(_plane_off(b, rb, 1), _CHUNK)], q1_v, semi),
            pltpu.async_copy(
                bary_hbm.at[pl.ds(_plane_off(b, rb, 2), _CHUNK)], q2_v, semi),
        ]
        for h in ins:
            h.wait()

        @pl.when(ci > 0)
        def _():
            pltpu.make_async_copy(
                out_v, vto_hbm.at[pl.ds(rb * _CHUNK * 2, _CHUNK * 2)], semo
            ).wait()

        def vec_body(mi, carry2):
            for half in range(2):
                l = mi * 32 + half * 16
                f3 = idx_v[pl.ds(l, 16)] * 3
                t0 = plsc.load_gather(vti_v, [f3]) * 2
                t1 = plsc.load_gather(vti_v, [f3 + 1]) * 2
                t2 = plsc.load_gather(vti_v, [f3 + 2]) * 2
                q0 = q0_v[pl.ds(l, 16)]
                q1 = q1_v[pl.ds(l, 16)]
                q2 = q2_v[pl.ds(l, 16)]
                vtx = (plsc.load_gather(vt_v, [t0]) * q0
                       + plsc.load_gather(vt_v, [t1]) * q1
                       + plsc.load_gather(vt_v, [t2]) * q2) * 2.0 - 1.0
                vty = (plsc.load_gather(vt_v, [t0 + 1]) * q0
                       + plsc.load_gather(vt_v, [t1 + 1]) * q1
                       + plsc.load_gather(vt_v, [t2 + 1]) * q2) * 2.0 - 1.0
                vtoff = ((l >> 7) & 7) * 1024 + (l >> 10) * 256 + (l & 127)
                out_v[pl.ds(vtoff, 16)] = vtx
                out_v[pl.ds(vtoff + 128, 16)] = vty
            return carry2

        lax.fori_loop(0, _CHUNK // 32, vec_body, 0)
        pltpu.async_copy(
            out_v, vto_hbm.at[pl.ds(rb * _CHUNK * 2, _CHUNK * 2)], semo
        )
        return carry

    lax.fori_loop(0, 8, chunk_body, 0)
    pltpu.make_async_copy(
        out_v, vto_hbm.at[pl.ds((rb0 + 7) * _CHUNK * 2, _CHUNK * 2)], semo
    ).wait()


@functools.partial(
    pl.kernel,
    mesh=_mesh,
    compiler_params=_params,
    out_type=jax.ShapeDtypeStruct((_NPIX * 3,), jnp.float32),
    scratch_types=[
        pltpu.VMEM((_V * 3,), jnp.float32),
        pltpu.VMEM((_F * 3,), jnp.int32),
        pltpu.VMEM((_CHUNK,), jnp.int32),
        pltpu.VMEM((_CHUNK,), jnp.float32),
        pltpu.VMEM((_CHUNK,), jnp.float32),
        pltpu.VMEM((_CHUNK,), jnp.float32),
        pltpu.VMEM((_CHUNK,), jnp.float32),
        pltpu.VMEM((_CHUNK,), jnp.float32),
        pltpu.VMEM((_CHUNK,), jnp.float32),
        pltpu.SemaphoreType.DMA,
        pltpu.SemaphoreType.DMA,
        pltpu.SemaphoreType.DMA,
    ],
)
def _normals(vn_hbm, vi_hbm, idx_hbm, bary_hbm, vno_hbm,
             vn_v, vi_v, idx_v, q0_v, q1_v, q2_v, n0_v, n1_v, n2_v,
             semt, semi, semo):
    b, y0, rb0 = _worker()
    loads = [
        pltpu.async_copy(vn_hbm.at[b], vn_v, semt),
        pltpu.async_copy(vi_hbm, vi_v, semt),
    ]
    for h in loads:
        h.wait()

    def outs(rb):
        return [
            (n0_v, _plane_off(b, rb, 0)),
            (n1_v, _plane_off(b, rb, 1)),
            (n2_v, _plane_off(b, rb, 2)),
        ]

    def chunk_body(ci, carry):
        rb = rb0 + ci
        ins = [
            pltpu.async_copy(idx_hbm.at[pl.ds(rb * _CHUNK, _CHUNK)], idx_v, semi),
            pltpu.async_copy(
                bary_hbm.at[pl.ds(_plane_off(b, rb, 0), _CHUNK)], q0_v, semi),
            pltpu.async_copy(
                bary_hbm.at[pl.ds(_plane_off(b, rb, 1), _CHUNK)], q1_v, semi),
            pltpu.async_copy(
                bary_hbm.at[pl.ds(_plane_off(b, rb, 2), _CHUNK)], q2_v, semi),
        ]
        for h in ins:
            h.wait()

        @pl.when(ci > 0)
        def _():
            for src, off in outs(rb):
                pltpu.make_async_copy(
                    src, vno_hbm.at[pl.ds(off, _CHUNK)], semo
                ).wait()

        def vec_body(mi, carry2):
            for half in range(2):
                l = mi * 32 + half * 16
                f3 = idx_v[pl.ds(l, 16)] * 3
                i0 = plsc.load_gather(vi_v, [f3]) * 3
                i1 = plsc.load_gather(vi_v, [f3 + 1]) * 3
                i2 = plsc.load_gather(vi_v, [f3 + 2]) * 3
                q0 = q0_v[pl.ds(l, 16)]
                q1 = q1_v[pl.ds(l, 16)]
                q2 = q2_v[pl.ds(l, 16)]
                n0_v[pl.ds(l, 16)] = (
                    plsc.load_gather(vn_v, [i0]) * q0
                    + plsc.load_gather(vn_v, [i1]) * q1
                    + plsc.load_gather(vn_v, [i2]) * q2)
                n1_v[pl.ds(l, 16)] = (
                    plsc.load_gather(vn_v, [i0 + 1]) * q0
                    + plsc.load_gather(vn_v, [i1 + 1]) * q1
                    + plsc.load_gather(vn_v, [i2 + 1]) * q2)
                n2_v[pl.ds(l, 16)] = (
                    plsc.load_gather(vn_v, [i0 + 2]) * q0
                    + plsc.load_gather(vn_v, [i1 + 2]) * q1
                    + plsc.load_gather(vn_v, [i2 + 2]) * q2)
            return carry2

        lax.fori_loop(0, _CHUNK // 32, vec_body, 0)
        for src, off in outs(rb):
            pltpu.async_copy(src, vno_hbm.at[pl.ds(off, _CHUNK)], semo)
        return carry

    lax.fori_loop(0, 8, chunk_body, 0)
    for src, off in outs(rb0 + 7):
        pltpu.make_async_copy(src, vno_hbm.at[pl.ds(off, _CHUNK)], semo).wait()


def kernel(v2d, vt, vn, vi, vti, index_img):
    vi_f = vi.reshape(-1)
    vti_f = vti.reshape(-1)
    v2d_f = v2d.reshape(_B, _V * 3)
    vn_f = vn.reshape(_B, _V * 3)
    vt_f = vt.reshape(-1)
    idx_tiled = index_img.reshape(_B, 64, 8, 4, 128)
    idx_tiled = idx_tiled.transpose(0, 1, 3, 2, 4).reshape(-1)
    depth, bary = _geom(v2d_f, vi_f, idx_tiled)
    vto = _texco(vt_f, vti_f, idx_tiled, bary)
    vno = _normals(vn_f, vi_f, idx_tiled, bary)
    depth = depth.reshape(_B, 64, 4, 8, 128).transpose(0, 1, 3, 2, 4)
    depth = depth.reshape(_B, _H, _W)
    bary = bary.reshape(_B, 3, 64, 4, 8, 128).transpose(0, 2, 4, 3, 5, 1)
    bary = bary.reshape(_B, _H, _W, 3)
    vno = vno.reshape(_B, 3, 64, 4, 8, 128).transpose(0, 2, 4, 3, 5, 1)
    vno = vno.reshape(_B, _H, _W, 3)
    vto = vto.reshape(_B, _H, 4, 2, 128).transpose(0, 1, 2, 4, 3)
    vto = vto.reshape(_B, _H, _W, 2)
    return depth, bary, vto, vno


# speedup vs baseline: 378.5046x; 1.0316x over previous
import functools

import jax
import jax.numpy as jnp
from jax import lax
from jax.experimental import pallas as pl
from jax.experimental.pallas import tpu as pltpu
from jax.experimental.pallas import tpu_sc as plsc

_H, _W = 512, 512
_B, _V, _F, _VT = 4, 10000, 20000, 12000
_NW = 32
_NPIX = _B * _H * _W
_CHUNK = 4096

_mesh = plsc.VectorSubcoreMesh(core_axis_name="c", subcore_axis_name="s")
_params = pltpu.CompilerParams(
    needs_layout_passes=False, use_tc_tiling_on_sc=False
)


def _eclamp(x):
    return jnp.where(x < 0, jnp.minimum(x, -1e-8), jnp.maximum(x, 1e-8))


def _worker():
    wid = lax.axis_index("s") * 2 + lax.axis_index("c")
    b = wid // 8
    y0 = (wid % 8) * 64
    rb0 = b * 64 + (y0 >> 3)
    return b, y0, rb0


def _plane_off(b, rb, c):
    return ((b * 2 + c) * 64 + rb) * _CHUNK


@functools.partial(
    pl.kernel,
    mesh=_mesh,
    compiler_params=_params,
    out_type=(
        jax.ShapeDtypeStruct((_NPIX,), jnp.float32),
        jax.ShapeDtypeStruct((_NPIX * 3,), jnp.float32),
    ),
    scratch_types=[
        pltpu.VMEM((_V * 3,), jnp.float32),
        pltpu.VMEM((_F * 3,), jnp.int32),
        pltpu.VMEM((_CHUNK,), jnp.int32),
        pltpu.VMEM((_CHUNK,), jnp.int32),
        pltpu.VMEM((_CHUNK,), jnp.float32),
        pltpu.VMEM((_CHUNK,), jnp.float32),
        pltpu.VMEM((_CHUNK,), jnp.float32),
        pltpu.VMEM((_CHUNK,), jnp.float32),
        pltpu.SemaphoreType.DMA,
        pltpu.SemaphoreType.DMA,
        pltpu.SemaphoreType.DMA,
    ],
)
def _geom(v2d_hbm, vi_hbm, idx_hbm, depth_hbm, bary_hbm,
          v2d_v, vi_v, idx_a, idx_b, dep_v, q0_v, q1_v, q2_v,
          semt, semi, semo):
    b, y0, rb0 = _worker()
    loads = [
        pltpu.async_copy(v2d_hbm.at[b], v2d_v, semt),
        pltpu.async_copy(vi_hbm, vi_v, semt),
    ]
    for h in loads:
        h.wait()
    iota = lax.iota(jnp.int32, 16)

    def outs(rb):
        return [
            (dep_v, depth_hbm, rb * _CHUNK),
            (q0_v, bary_hbm, _plane_off(b, rb, 0)),
            (q1_v, bary_hbm, _plane_off(b, rb, 1)),
            (q2_v, bary_hbm, _plane_off(b, rb, 2)),
        ]

    idx_bufs = (idx_a, idx_b)
    pltpu.async_copy(idx_hbm.at[pl.ds(rb0 * _CHUNK, _CHUNK)], idx_a, semi)

    def chunk_pair(t, carry):
      for par in range(2):
        ci = t * 2 + par
        yc = y0 + ci * 8
        rb = rb0 + ci
        idx_v = idx_bufs[par]
        nxt = idx_bufs[1 - par]
        pltpu.make_async_copy(
            idx_hbm.at[pl.ds(rb * _CHUNK, _CHUNK)], idx_v, semi
        ).wait()

        @pl.when(ci < 7)
        def _():
            pltpu.async_copy(
                idx_hbm.at[pl.ds((rb + 1) * _CHUNK, _CHUNK)], nxt, semi
            )

        @pl.when(ci > 0)
        def _():
            for src, dst, off in outs(rb):
                pltpu.make_async_copy(src, dst.at[pl.ds(off, _CHUNK)], semo).wait()

        def vec_body(mi, carry2):
            for half in range(2):
                l = mi * 32 + half * 16
                f3 = idx_v[pl.ds(l, 16)] * 3
                i0 = plsc.load_gather(vi_v, [f3]) * 3
                i1 = plsc.load_gather(vi_v, [f3 + 1]) * 3
                i2 = plsc.load_gather(vi_v, [f3 + 2]) * 3
                v0x = plsc.load_gather(v2d_v, [i0])
                v0y = plsc.load_gather(v2d_v, [i0 + 1])
                v0z = plsc.load_gather(v2d_v, [i0 + 2])
                v1x = plsc.load_gather(v2d_v, [i1])
                v1y = plsc.load_gather(v2d_v, [i1 + 1])
                v1z = plsc.load_gather(v2d_v, [i1 + 2])
                v2x = plsc.load_gather(v2d_v, [i2])
                v2y = plsc.load_gather(v2d_v, [i2 + 1])
                v2z = plsc.load_gather(v2d_v, [i2 + 2])
                xb = ((l >> 10) << 7) | (l & 127)
                x = (iota + xb).astype(jnp.float32)
                y = (yc + ((l >> 7) & 7)).astype(jnp.float32)
                e1x = v1x - v0x
                e1y = v1y - v0y
                e2x = v2x - v0x
                e2y = v2y - v0y
                den = _eclamp(e1x * e2y - e1y * e2x)
                w0 = 1.0 / _eclamp(v0z)
                w1 = 1.0 / _eclamp(v1z)
                w2 = 1.0 / _eclamp(v2z)
                px = x - v0x
                py = y - v0y
                l1 = (px * e2y - py * e2x) / den
                l2 = (py * e1x - px * e1y) / den
                lam0 = 1.0 - l1 - l2
                u0 = w0 * lam0
                u1 = w1 * l1
                u2 = w2 * l2
                zi = 1.0 / _eclamp(u0 + u1 + u2)
                dep_v[pl.ds(l, 16)] = zi
                q0_v[pl.ds(l, 16)] = u0 * zi
                q1_v[pl.ds(l, 16)] = u1 * zi
                q2_v[pl.ds(l, 16)] = u2 * zi
            return carry2

        lax.fori_loop(0, _CHUNK // 32, vec_body, 0)
        for src, dst, off in outs(rb):
            pltpu.async_copy(src, dst.at[pl.ds(off, _CHUNK)], semo)
      return carry

    lax.fori_loop(0, 4, chunk_pair, 0)
    for src, dst, off in outs(rb0 + 7):
        pltpu.make_async_copy(src, dst.at[pl.ds(off, _CHUNK)], semo).wait()


@functools.partial(
    pl.kernel,
    mesh=_mesh,
    compiler_params=_params,
    out_type=jax.ShapeDtypeStruct((_NPIX * 2,), jnp.float32),
    scratch_types=[
        pltpu.VMEM((_VT * 2,), jnp.float32),
        pltpu.VMEM((_F * 3,), jnp.int32),
        pltpu.VMEM((_CHUNK,), jnp.int32),
        pltpu.VMEM((_CHUNK,), jnp.int32),
        pltpu.VMEM((_CHUNK,), jnp.float32),
        pltpu.VMEM((_CHUNK,), jnp.float32),
        pltpu.VMEM((_CHUNK,), jnp.float32),
        pltpu.VMEM((_CHUNK,), jnp.float32),
        pltpu.VMEM((_CHUNK,), jnp.float32),
        pltpu.VMEM((_CHUNK,), jnp.float32),
        pltpu.VMEM((_CHUNK * 2,), jnp.float32),
        pltpu.SemaphoreType.DMA,
        pltpu.SemaphoreType.DMA,
        pltpu.SemaphoreType.DMA,
    ],
)
def _texco(vt_hbm, vti_hbm, idx_hbm, bary_hbm, vto_hbm,
           vt_v, vti_v, idx_a, idx_b, qa0, qa1, qa2, qb0, qb1, qb2,
           out_v, semt, semi, semo):
    b, y0, rb0 = _worker()
    loads = [
        pltpu.async_copy(vt_hbm, vt_v, semt),
        pltpu.async_copy(vti_hbm, vti_v, semt),
    ]
    for h in loads:
        h.wait()

    insets = ((idx_a, qa0, qa1, qa2), (idx_b, qb0, qb1, qb2))

    def fire_ins(rb, bufs):
        pltpu.async_copy(idx_hbm.at[pl.ds(rb * _CHUNK, _CHUNK)], bufs[0], semi)
        for c in range(3):
            pltpu.async_copy(
                bary_hbm.at[pl.ds(_plane_off(b, rb, c), _CHUNK)],
                bufs[1 + c], semi)

    def wait_ins(rb, bufs):
        pltpu.make_async_copy(
            idx_hbm.at[pl.ds(rb * _CHUNK, _CHUNK)], bufs[0], semi).wait()
        for c in range(3):
            pltpu.make_async_copy(
                bary_hbm.at[pl.ds(_plane_off(b, rb, c), _CHUNK)],
                bufs[1 + c], semi).wait()

    fire_ins(rb0, insets[0])

    def chunk_pair(t, carry):
      for par in range(2):
        ci = t * 2 + par
        rb = rb0 + ci
        idx_v, q0_v, q1_v, q2_v = insets[par]
        wait_ins(rb, insets[par])

        @pl.when(ci < 7)
        def _():
            fire_ins(rb + 1, insets[1 - par])

        @pl.when(ci > 0)
        def _():
            pltpu.make_async_copy(
                out_v, vto_hbm.at[pl.ds(rb * _CHUNK * 2, _CHUNK * 2)], semo
            ).wait()

        def vec_body(mi, carry2):
            for half in range(2):
                l = mi * 32 + half * 16
                f3 = idx_v[pl.ds(l, 16)] * 3
                t0 = plsc.load_gather(vti_v, [f3]) * 2
                t1 = plsc.load_gather(vti_v, [f3 + 1]) * 2
                t2 = plsc.load_gather(vti_v, [f3 + 2]) * 2
                q0 = q0_v[pl.ds(l, 16)]
                q1 = q1_v[pl.ds(l, 16)]
                q2 = q2_v[pl.ds(l, 16)]
                vtx = (plsc.load_gather(vt_v, [t0]) * q0
                       + plsc.load_gather(vt_v, [t1]) * q1
                       + plsc.load_gather(vt_v, [t2]) * q2) * 2.0 - 1.0
                vty = (plsc.load_gather(vt_v, [t0 + 1]) * q0
                       + plsc.load_gather(vt_v, [t1 + 1]) * q1
                       + plsc.load_gather(vt_v, [t2 + 1]) * q2) * 2.0 - 1.0
                vtoff = ((l >> 7) & 7) * 1024 + (l >> 10) * 256 + (l & 127)
                out_v[pl.ds(vtoff, 16)] = vtx
                out_v[pl.ds(vtoff + 128, 16)] = vty
            return carry2

        lax.fori_loop(0, _CHUNK // 32, vec_body, 0)
        pltpu.async_copy(
            out_v, vto_hbm.at[pl.ds(rb * _CHUNK * 2, _CHUNK * 2)], semo
        )
      return carry

    lax.fori_loop(0, 4, chunk_pair, 0)
    pltpu.make_async_copy(
        out_v, vto_hbm.at[pl.ds((rb0 + 7) * _CHUNK * 2, _CHUNK * 2)], semo
    ).wait()


@functools.partial(
    pl.kernel,
    mesh=_mesh,
    compiler_params=_params,
    out_type=jax.ShapeDtypeStruct((_NPIX * 3,), jnp.float32),
    scratch_types=[
        pltpu.VMEM((_V * 3,), jnp.float32),
        pltpu.VMEM((_F * 3,), jnp.int32),
        pltpu.VMEM((_CHUNK,), jnp.int32),
        pltpu.VMEM((_CHUNK,), jnp.float32),
        pltpu.VMEM((_CHUNK,), jnp.float32),
        pltpu.VMEM((_CHUNK,), jnp.float32),
        pltpu.VMEM((_CHUNK,), jnp.float32),
        pltpu.VMEM((_CHUNK,), jnp.float32),
        pltpu.VMEM((_CHUNK,), jnp.float32),
        pltpu.SemaphoreType.DMA,
        pltpu.SemaphoreType.DMA,
        pltpu.SemaphoreType.DMA,
    ],
)
def _normals(vn_hbm, vi_hbm, idx_hbm, bary_hbm, vno_hbm,
             vn_v, vi_v, idx_v, q0_v, q1_v, q2_v, n0_v, n1_v, n2_v,
             semt, semi, semo):
    b, y0, rb0 = _worker()
    loads = [
        pltpu.async_copy(vn_hbm.at[b], vn_v, semt),
        pltpu.async_copy(vi_hbm, vi_v, semt),
    ]
    for h in loads:
        h.wait()

    def outs(rb):
        return [
            (n0_v, _plane_off(b, rb, 0)),
            (n1_v, _plane_off(b, rb, 1)),
            (n2_v, _plane_off(b, rb, 2)),
        ]

    def chunk_body(ci, carry):
        rb = rb0 + ci
        ins = [
            pltpu.async_copy(idx_hbm.at[pl.ds(rb * _CHUNK, _CHUNK)], idx_v, semi),
            pltpu.async_copy(
                bary_hbm.at[pl.ds(_plane_off(b, rb, 0), _CHUNK)], q0_v, semi),
            pltpu.async_copy(
                bary_hbm.at[pl.ds(_plane_off(b, rb, 1), _CHUNK)], q1_v, semi),
            pltpu.async_copy(
                bary_hbm.at[pl.ds(_plane_off(b, rb, 2), _CHUNK)], q2_v, semi),
        ]
        for h in ins:
            h.wait()

        @pl.when(ci > 0)
        def _():
            for src, off in outs(rb):
                pltpu.make_async_copy(
                    src, vno_hbm.at[pl.ds(off, _CHUNK)], semo
                ).wait()

        def vec_body(mi, carry2):
            for half in range(2):
                l = mi * 32 + half * 16
                f3 = idx_v[pl.ds(l, 16)] * 3
                i0 = plsc.load_gather(vi_v, [f3]) * 3
                i1 = plsc.load_gather(vi_v, [f3 + 1]) * 3
                i2 = plsc.load_gather(vi_v, [f3 + 2]) * 3
                q0 = q0_v[pl.ds(l, 16)]
                q1 = q1_v[pl.ds(l, 16)]
                q2 = q2_v[pl.ds(l, 16)]
                n0_v[pl.ds(l, 16)] = (
                    plsc.load_gather(vn_v, [i0]) * q0
                    + plsc.load_gather(vn_v, [i1]) * q1
                    + plsc.load_gather(vn_v, [i2]) * q2)
                n1_v[pl.ds(l, 16)] = (
                    plsc.load_gather(vn_v, [i0 + 1]) * q0
                    + plsc.load_gather(vn_v, [i1 + 1]) * q1
                    + plsc.load_gather(vn_v, [i2 + 1]) * q2)
                n2_v[pl.ds(l, 16)] = (
                    plsc.load_gather(vn_v, [i0 + 2]) * q0
                    + plsc.load_gather(vn_v, [i1 + 2]) * q1
                    + plsc.load_gather(vn_v, [i2 + 2]) * q2)
            return carry2

        lax.fori_loop(0, _CHUNK // 32, vec_body, 0)
        for src, off in outs(rb):
            pltpu.async_copy(src, vno_hbm.at[pl.ds(off, _CHUNK)], semo)
        return carry

    lax.fori_loop(0, 8, chunk_body, 0)
    for src, off in outs(rb0 + 7):
        pltpu.make_async_copy(src, vno_hbm.at[pl.ds(off, _CHUNK)], semo).wait()


def kernel(v2d, vt, vn, vi, vti, index_img):
    vi_f = vi.reshape(-1)
    vti_f = vti.reshape(-1)
    v2d_f = v2d.reshape(_B, _V * 3)
    vn_f = vn.reshape(_B, _V * 3)
    vt_f = vt.reshape(-1)
    idx_tiled = index_img.reshape(_B, 64, 8, 4, 128)
    idx_tiled = idx_tiled.transpose(0, 1, 3, 2, 4).reshape(-1)
    depth, bary = _geom(v2d_f, vi_f, idx_tiled)
    vto = _texco(vt_f, vti_f, idx_tiled, bary)
    vno = _normals(vn_f, vi_f, idx_tiled, bary)
    depth = depth.reshape(_B, 64, 4, 8, 128).transpose(0, 1, 3, 2, 4)
    depth = depth.reshape(_B, _H, _W)
    bary = bary.reshape(_B, 3, 64, 4, 8, 128).transpose(0, 2, 4, 3, 5, 1)
    bary = bary.reshape(_B, _H, _W, 3)
    vno = vno.reshape(_B, 3, 64, 4, 8, 128).transpose(0, 2, 4, 3, 5, 1)
    vno = vno.reshape(_B, _H, _W, 3)
    vto = vto.reshape(_B, _H, 4, 2, 128).transpose(0, 1, 2, 4, 3)
    vto = vto.reshape(_B, _H, _W, 2)
    return depth, bary, vto, vno
